# baseline (speedup 1.0000x reference)
import functools

import jax
import jax.numpy as jnp
from jax import lax
from jax.experimental import pallas as pl
from jax.experimental.pallas import tpu as pltpu
from jax.experimental.pallas import tpu_sc as plsc

D = 64
BATCH = 4096
SEQ = 50
NC = 2
NS = 16
NW = NC * NS
ROWS_PER_TILE = BATCH // NW
IDX_PAD = 56
IDX_BLOCK = ROWS_PER_TILE * IDX_PAD
NBUF = 8
NGROUPS = ROWS_PER_TILE // NBUF

_mesh = plsc.VectorSubcoreMesh(core_axis_name="c", subcore_axis_name="s")


@functools.partial(
    pl.kernel,
    mesh=_mesh,
    out_type=jax.ShapeDtypeStruct((BATCH, SEQ, D), jnp.float32),
    scratch_types=[
        pltpu.VMEM((IDX_BLOCK,), jnp.int32),
        pltpu.VMEM((NBUF, SEQ, D), jnp.float32),
        pltpu.SemaphoreType.DMA((NBUF,)),
        pltpu.SemaphoreType.DMA((NBUF,)),
    ],
    compiler_params=pltpu.CompilerParams(use_tc_tiling_on_sc=False),
)
def _gather(idx_hbm, table_hbm, out_hbm, idx_v, rows_v, sem_g, sem_o):
    wid = lax.axis_index("s") * NC + lax.axis_index("c")
    base = wid * ROWS_PER_TILE
    pltpu.sync_copy(idx_hbm.at[wid], idx_v)

    def group(g, carry):
        j0 = g * NBUF
        gathers = []
        for b in range(NBUF):
            @pl.when(g > 0)
            def _reclaim(b=b):
                pltpu.make_async_copy(
                    rows_v.at[b], out_hbm.at[base], sem_o.at[b]
                ).wait()

            gathers.append(
                pltpu.async_copy(
                    table_hbm.at[idx_v.at[pl.ds((j0 + b) * IDX_PAD, SEQ)]],
                    rows_v.at[b],
                    sem_g.at[b],
                )
            )
        for b in range(NBUF):
            gathers[b].wait()
            pltpu.async_copy(rows_v.at[b], out_hbm.at[base + j0 + b], sem_o.at[b])
        return carry

    lax.fori_loop(0, NGROUPS, group, 0)

    for b in range(NBUF):
        pltpu.make_async_copy(rows_v.at[b], out_hbm.at[base], sem_o.at[b]).wait()


def kernel(inputs, embedding):
    idx = jnp.pad(inputs, ((0, 0), (0, IDX_PAD - SEQ)))
    idx = idx.reshape(NW, IDX_BLOCK)
    return _gather(idx, embedding)


# speedup vs baseline: 1.2750x; 1.2750x over previous
import functools

import jax
import jax.numpy as jnp
from jax import lax
from jax.experimental import pallas as pl
from jax.experimental.pallas import tpu as pltpu
from jax.experimental.pallas import tpu_sc as plsc

D = 64
DP = 128
BATCH = 4096
SEQ = 50
SEQ_P = 56
NC = 2
NS = 16
NW = NC * NS
ROWS_PER_TILE = BATCH // NW
IDX_BLOCK = ROWS_PER_TILE * SEQ_P
NBUF = 8
NGROUPS = ROWS_PER_TILE // NBUF

_mesh = plsc.VectorSubcoreMesh(core_axis_name="c", subcore_axis_name="s")


@functools.partial(
    pl.kernel,
    mesh=_mesh,
    out_type=jax.ShapeDtypeStruct((BATCH, SEQ_P, DP), jnp.float32),
    scratch_types=[
        pltpu.VMEM((IDX_BLOCK,), jnp.int32),
        pltpu.VMEM((NBUF, SEQ, DP), jnp.float32),
        pltpu.SemaphoreType.DMA((NBUF,)),
        pltpu.SemaphoreType.DMA((NBUF,)),
    ],
    compiler_params=pltpu.CompilerParams(use_tc_tiling_on_sc=False),
)
def _gather(idx_hbm, table_hbm, out_hbm, idx_v, rows_v, sem_g, sem_o):
    wid = lax.axis_index("s") * NC + lax.axis_index("c")
    base = wid * ROWS_PER_TILE
    pltpu.sync_copy(idx_hbm.at[wid], idx_v)

    def group(g, carry):
        j0 = g * NBUF
        gathers = []
        for b in range(NBUF):
            @pl.when(g > 0)
            def _reclaim(b=b):
                pltpu.make_async_copy(
                    rows_v.at[b], out_hbm.at[base, pl.ds(0, SEQ)], sem_o.at[b]
                ).wait()

            gathers.append(
                pltpu.async_copy(
                    table_hbm.at[idx_v.at[pl.ds((j0 + b) * SEQ_P, SEQ)]],
                    rows_v.at[b],
                    sem_g.at[b],
                )
            )
        for b in range(NBUF):
            gathers[b].wait()
            pltpu.async_copy(
                rows_v.at[b], out_hbm.at[base + j0 + b, pl.ds(0, SEQ)], sem_o.at[b]
            )
        return carry

    lax.fori_loop(0, NGROUPS, group, 0)

    for b in range(NBUF):
        pltpu.make_async_copy(
            rows_v.at[b], out_hbm.at[base, pl.ds(0, SEQ)], sem_o.at[b]
        ).wait()


def kernel(inputs, embedding):
    idx = jnp.pad(inputs, ((0, 0), (0, SEQ_P - SEQ)))
    idx = idx.reshape(NW, IDX_BLOCK)
    table = jnp.pad(embedding, ((0, 0), (0, DP - D)))
    out = _gather(idx, table)
    return lax.slice(out, (0, 0, 0), (BATCH, SEQ, D))


# speedup vs baseline: 1.5771x; 1.2369x over previous
import functools

import jax
import jax.numpy as jnp
from jax import lax
from jax.experimental import pallas as pl
from jax.experimental.pallas import tpu as pltpu
from jax.experimental.pallas import tpu_sc as plsc

D = 64
DP = 128
BATCH = 4096
SEQ = 50
SEQ_P = 56
NC = 2
NS = 16
NW = NC * NS
ROWS_PER_TILE = BATCH // NW
IDX_BLOCK = ROWS_PER_TILE * SEQ_P
NBUF = 8
NGROUPS = ROWS_PER_TILE // NBUF

_mesh = plsc.VectorSubcoreMesh(core_axis_name="c", subcore_axis_name="s")


@functools.partial(
    pl.kernel,
    mesh=_mesh,
    out_type=jax.ShapeDtypeStruct((BATCH, SEQ_P, DP), jnp.float32),
    scratch_types=[
        pltpu.VMEM((IDX_BLOCK,), jnp.int32),
        pltpu.VMEM((NBUF, SEQ, D), jnp.float32),
        pltpu.SemaphoreType.DMA((NBUF,)),
        pltpu.SemaphoreType.DMA((NBUF,)),
    ],
    compiler_params=pltpu.CompilerParams(use_tc_tiling_on_sc=False),
)
def _gather(idx_hbm, table_hbm, out_hbm, idx_v, rows_v, sem_g, sem_o):
    wid = lax.axis_index("s") * NC + lax.axis_index("c")
    base = wid * ROWS_PER_TILE
    pltpu.sync_copy(idx_hbm.at[wid], idx_v)

    def group(g, carry):
        j0 = g * NBUF
        gathers = []
        for b in range(NBUF):
            @pl.when(g > 0)
            def _reclaim(b=b):
                pltpu.make_async_copy(
                    rows_v.at[b],
                    out_hbm.at[base, pl.ds(0, SEQ), pl.ds(0, D)],
                    sem_o.at[b],
                ).wait()

            gathers.append(
                pltpu.async_copy(
                    table_hbm.at[idx_v.at[pl.ds((j0 + b) * SEQ_P, SEQ)]],
                    rows_v.at[b],
                    sem_g.at[b],
                )
            )
        for b in range(NBUF):
            gathers[b].wait()
            pltpu.async_copy(
                rows_v.at[b],
                out_hbm.at[base + j0 + b, pl.ds(0, SEQ), pl.ds(0, D)],
                sem_o.at[b],
            )
        return carry

    lax.fori_loop(0, NGROUPS, group, 0)

    for b in range(NBUF):
        pltpu.make_async_copy(
            rows_v.at[b],
            out_hbm.at[base, pl.ds(0, SEQ), pl.ds(0, D)],
            sem_o.at[b],
        ).wait()


def kernel(inputs, embedding):
    idx = jnp.pad(inputs * 2, ((0, 0), (0, SEQ_P - SEQ)))
    idx = idx.reshape(NW, IDX_BLOCK)
    table = jnp.pad(embedding, ((0, 0), (0, DP - D))).reshape(2 * embedding.shape[0], D)
    out = _gather(idx, table)
    return lax.slice(out, (0, 0, 0), (BATCH, SEQ, D))
